# Initial kernel scaffold; baseline (speedup 1.0000x reference)
import jax
import jax.numpy as jnp
from jax.experimental import pallas as pl


def kernel(x, W_enc, b_enc, W_dec, b_dec):
    raise NotImplementedError("write your pallas kernel here")


# speedup vs baseline: 11.7570x; 11.7570x over previous
import functools

import jax
import jax.numpy as jnp
from jax.experimental import pallas as pl
from jax.experimental.pallas import tpu as pltpu

TOPK = 64


def _encode_body(x_ref, w_ref, benc_ref, bdec_ref, out_ref):
    x = x_ref[...] - bdec_ref[...][None, :]
    pre = jax.lax.dot_general(
        x, w_ref[...], (((1,), (1,)), ((), ())),
        preferred_element_type=jnp.float32)
    out_ref[...] = jnp.maximum(pre + benc_ref[...][None, :], 0.0)


def _mask_body(post_ref, out_ref, *, k):
    v = post_ref[...]
    vi = jax.lax.bitcast_convert_type(v, jnp.int32)
    lo = jnp.zeros((v.shape[0], 1), jnp.int32)
    for bit in range(30, -1, -1):
        t = lo | (1 << bit)
        cnt = jnp.sum((vi >= t).astype(jnp.int32), axis=1, keepdims=True)
        lo = jnp.where(cnt >= k, t, lo)
    out_ref[...] = jnp.where(vi >= lo, v, 0.0)


def _decode_body(m_ref, w_ref, bdec_ref, out_ref):
    kk = pl.program_id(1)

    @pl.when(kk == 0)
    def _init():
        out_ref[...] = jnp.broadcast_to(bdec_ref[...][None, :], out_ref.shape)

    out_ref[...] += jax.lax.dot_general(
        m_ref[...], w_ref[...], (((1,), (1,)), ((), ())),
        preferred_element_type=jnp.float32)


def _run(x, W_enc, b_enc, W_dec, b_dec, *, k, tb, tf, tb2, tb3, tfk,
         interpret=False):
    B, D = x.shape
    F = W_enc.shape[0]

    post = pl.pallas_call(
        _encode_body,
        grid=(F // tf, B // tb),
        in_specs=[
            pl.BlockSpec((tb, D), lambda f, b: (b, 0)),
            pl.BlockSpec((tf, D), lambda f, b: (f, 0)),
            pl.BlockSpec((tf,), lambda f, b: (f,)),
            pl.BlockSpec((D,), lambda f, b: (0,)),
        ],
        out_specs=pl.BlockSpec((tb, tf), lambda f, b: (b, f)),
        out_shape=jax.ShapeDtypeStruct((B, F), jnp.float32),
        interpret=interpret,
    )(x, W_enc, b_enc, b_dec)

    masked = pl.pallas_call(
        functools.partial(_mask_body, k=k),
        grid=(B // tb2,),
        in_specs=[pl.BlockSpec((tb2, F), lambda b: (b, 0))],
        out_specs=pl.BlockSpec((tb2, F), lambda b: (b, 0)),
        out_shape=jax.ShapeDtypeStruct((B, F), jnp.float32),
        interpret=interpret,
    )(post)

    x_hat = pl.pallas_call(
        _decode_body,
        grid=(B // tb3, F // tfk),
        in_specs=[
            pl.BlockSpec((tb3, tfk), lambda i, kk: (i, kk)),
            pl.BlockSpec((D, tfk), lambda i, kk: (0, kk)),
            pl.BlockSpec((D,), lambda i, kk: (0,)),
        ],
        out_specs=pl.BlockSpec((tb3, D), lambda i, kk: (i, 0)),
        out_shape=jax.ShapeDtypeStruct((B, D), jnp.float32),
        compiler_params=pltpu.CompilerParams(
            dimension_semantics=("arbitrary", "arbitrary")),
        interpret=interpret,
    )(masked, W_dec, b_dec)
    return x_hat


def kernel(x, W_enc, b_enc, W_dec, b_dec):
    return _run(x, W_enc, b_enc, W_dec, b_dec,
                k=TOPK, tb=512, tf=2048, tb2=128, tb3=512, tfk=2048)
